# baseline (speedup 1.0000x reference)
import functools

import jax
import jax.numpy as jnp
from jax import lax
from jax.experimental import pallas as pl
from jax.experimental.pallas import tpu as pltpu
from jax.experimental.pallas import tpu_sc as plsc

_B = 4096
_N = 200
_H = 64
_BB = 256


def _gather_uemb(weight, idx):
    info = plsc.get_sparse_core_info()
    nc, ns = info.num_cores, info.num_subcores
    nw = nc * ns
    b_per_w = _B // nw
    mesh = plsc.VectorSubcoreMesh(core_axis_name="c", subcore_axis_name="s")

    @functools.partial(
        pl.kernel,
        mesh=mesh,
        out_type=jax.ShapeDtypeStruct((_B, _H), jnp.float32),
        scratch_types=[
            pltpu.VMEM((b_per_w,), jnp.int32),
            pltpu.VMEM((b_per_w, _H), jnp.float32),
            pltpu.SemaphoreType.DMA,
        ],
        compiler_params=pltpu.CompilerParams(use_tc_tiling_on_sc=False),
    )
    def gather_k(table_hbm, idx_hbm, out_hbm, idx_v, rows_v, sem):
        wid = lax.axis_index("s") * nc + lax.axis_index("c")
        base = wid * b_per_w
        pltpu.sync_copy(idx_hbm.at[pl.ds(base, b_per_w)], idx_v)
        pltpu.async_copy(table_hbm.at[idx_v], rows_v, sem).wait()
        pltpu.sync_copy(rows_v, out_hbm.at[pl.ds(base, b_per_w)])

    return gather_k(weight, idx)


def _bmm(iemb, uemb):
    def body(x_ref, u_ref, o_ref):
        o_ref[...] = jnp.sum(x_ref[...] * u_ref[...][:, None, :], axis=2)

    return pl.pallas_call(
        body,
        grid=(_B // _BB,),
        in_specs=[
            pl.BlockSpec((_BB, _N, _H), lambda i: (i, 0, 0)),
            pl.BlockSpec((_BB, _H), lambda i: (i, 0)),
        ],
        out_specs=pl.BlockSpec((_BB, _N), lambda i: (i, 0)),
        out_shape=jax.ShapeDtypeStruct((_B, _N), jnp.float32),
    )(iemb, uemb)


def kernel(userid_input, iemb, uembedding_weight):
    idx = userid_input.reshape(-1)
    uemb = _gather_uemb(uembedding_weight, idx)
    return _bmm(iemb, uemb)
